# baseline (speedup 1.0000x reference)
import functools

import jax
import jax.numpy as jnp
from jax import lax
from jax.experimental import pallas as pl
from jax.experimental.pallas import tpu as pltpu
from jax.experimental.pallas import tpu_sc as plsc

_B = 16384
_D = 32
_NC = 2
_NS = 16
_NW = _NC * _NS
_N = _B // _NW
_CHUNK = 128
_NCHUNK = _N // _CHUNK
_SIZES = (1000000, 100000, 1000, 100000)


def _sc_body(idx_hbm, item_hbm, cat_hbm, cup_hbm, user_hbm, out_hbm,
             idx_v, rows_v, sem):
    wid = lax.axis_index("s") * _NC + lax.axis_index("c")
    base = wid * _N

    pltpu.sync_copy(idx_hbm.at[wid], idx_v)

    for c, size in enumerate(_SIZES):
        for j in range(_NCHUNK):
            def _mod_body(i, _, c=c, j=j, size=size):
                sl = pl.ds(i * 16, 16)
                idx_v[c, j, sl] = lax.rem(idx_v[c, j, sl], size)
                return 0
            lax.fori_loop(0, _CHUNK // 16, _mod_body, 0)

    tables = (item_hbm, cat_hbm, cup_hbm, user_hbm)
    for c, tab in enumerate(tables):
        for j in range(_NCHUNK):
            pltpu.async_copy(
                tab.at[idx_v.at[c, j]],
                rows_v.at[c, pl.ds(j * _CHUNK, _CHUNK)],
                sem)
    for c in range(4):
        pltpu.make_async_copy(
            tables[c].at[pl.ds(0, _N)], rows_v.at[c], sem).wait()

    for c in range(4):
        pltpu.sync_copy(rows_v.at[c],
                        out_hbm.at[pl.ds(base, _N), pl.ds(c * _D, _D)])


@jax.jit
def kernel(batch_input, item_table, category_table, cup_size_table,
           user_table):
    idx = batch_input.astype(jnp.int32)
    idx = idx.reshape(_NW, _NCHUNK, _CHUNK, 4).transpose(0, 3, 1, 2)

    mesh = plsc.VectorSubcoreMesh(core_axis_name="c", subcore_axis_name="s")
    run = functools.partial(
        pl.kernel,
        mesh=mesh,
        compiler_params=pltpu.CompilerParams(use_tc_tiling_on_sc=False),
        out_type=jax.ShapeDtypeStruct((_B, 4 * _D), jnp.float32),
        scratch_types=[
            pltpu.VMEM((4, _NCHUNK, _CHUNK), jnp.int32),
            pltpu.VMEM((4, _N, _D), jnp.float32),
            pltpu.SemaphoreType.DMA,
        ],
    )(_sc_body)
    return run(idx, item_table, category_table, cup_size_table, user_table)
